# baseline (speedup 1.0000x reference)
import functools

import jax
import jax.numpy as jnp
from jax import lax
from jax.experimental import pallas as pl
from jax.experimental.pallas import tpu as pltpu
from jax.experimental.pallas import tpu_sc as plsc

B = 1024
D = 16
NUM_CORES = 2
NUM_SUBCORES = 16
NW = NUM_CORES * NUM_SUBCORES
B_PER_W = B // NW


def _sc_gather(ent, ent_type, ent_emb, type_emb, ent_proj, type_proj):
    mesh = plsc.VectorSubcoreMesh(core_axis_name="c", subcore_axis_name="s")
    row_t = jax.ShapeDtypeStruct((B, D), jnp.float32)

    @functools.partial(
        pl.kernel,
        mesh=mesh,
        out_type=[row_t, row_t, row_t, row_t],
        scratch_types=[
            pltpu.VMEM((B_PER_W,), jnp.int32),
            pltpu.VMEM((B_PER_W,), jnp.int32),
            pltpu.VMEM((B_PER_W, D), jnp.float32),
            pltpu.VMEM((B_PER_W, D), jnp.float32),
            pltpu.VMEM((B_PER_W, D), jnp.float32),
            pltpu.VMEM((B_PER_W, D), jnp.float32),
            pltpu.SemaphoreType.DMA,
        ],
    )
    def gather_kernel(ent_hbm, etype_hbm, ee_hbm, te_hbm, ep_hbm, tp_hbm,
                      e_out, t_out, pe_out, pt_out,
                      idx_e, idx_t, e_v, t_v, pe_v, pt_v, sem):
        wid = lax.axis_index("s") * NUM_CORES + lax.axis_index("c")
        base = wid * B_PER_W
        sl = pl.ds(base, B_PER_W)
        pltpu.sync_copy(ent_hbm.at[sl], idx_e)
        pltpu.sync_copy(etype_hbm.at[sl], idx_t)
        copies = []
        for g in range(B_PER_W // 16):
            ve = idx_e[pl.ds(g * 16, 16)]
            vt = idx_t[pl.ds(g * 16, 16)]
            for k in range(16):
                r = g * 16 + k
                ie = ve[k]
                it = vt[k]
                copies.append(pltpu.async_copy(ee_hbm.at[ie], e_v.at[r], sem))
                copies.append(pltpu.async_copy(ep_hbm.at[ie], pe_v.at[r], sem))
                copies.append(pltpu.async_copy(te_hbm.at[it], t_v.at[r], sem))
                copies.append(pltpu.async_copy(tp_hbm.at[it], pt_v.at[r], sem))
        for c in copies:
            c.wait()
        pltpu.sync_copy(e_v, e_out.at[sl])
        pltpu.sync_copy(pe_v, pe_out.at[sl])
        pltpu.sync_copy(t_v, t_out.at[sl])
        pltpu.sync_copy(pt_v, pt_out.at[sl])

    return gather_kernel(ent, ent_type, ent_emb, type_emb, ent_proj, type_proj)


BI = 256


def _score_body(e_ref, t_ref, pet_ref, ptt_ref, out_ref):
    e = e_ref[...]
    t = t_ref[...]
    pet = pet_ref[...]
    ptt = ptt_ref[...]
    a = jax.lax.dot_general(e, pet, (((1,), (0,)), ((), ())),
                            preferred_element_type=jnp.float32,
                            precision=jax.lax.Precision.HIGHEST)
    c = jax.lax.dot_general(t, ptt, (((1,), (0,)), ((), ())),
                            preferred_element_type=jnp.float32,
                            precision=jax.lax.Precision.HIGHEST)
    diff = e - t
    acc = jnp.zeros((BI, B), jnp.float32)
    for d in range(D):
        term = diff[:, d:d + 1] + a * pet[d:d + 1, :] - c * ptt[d:d + 1, :]
        acc = acc + jnp.abs(term)
    out_ref[...] = acc


def _tc_score(e, t, pe_t, pt_t):
    return pl.pallas_call(
        _score_body,
        grid=(B // BI,),
        in_specs=[
            pl.BlockSpec((BI, D), lambda i: (i, 0)),
            pl.BlockSpec((BI, D), lambda i: (i, 0)),
            pl.BlockSpec((D, B), lambda i: (0, 0)),
            pl.BlockSpec((D, B), lambda i: (0, 0)),
        ],
        out_specs=pl.BlockSpec((BI, B), lambda i: (i, 0)),
        out_shape=jax.ShapeDtypeStruct((B, B), jnp.float32),
    )(e, t, pe_t, pt_t)


def kernel(ent, ent_type, ent_emb, type_emb, ent_proj, type_proj):
    e, t, pe, pt = _sc_gather(ent, ent_type, ent_emb, type_emb,
                              ent_proj, type_proj)
    return _tc_score(e, t, pe.T, pt.T)
